# Initial kernel scaffold; baseline (speedup 1.0000x reference)
import jax
import jax.numpy as jnp
from jax.experimental import pallas as pl


def kernel(x, edge_index, batch, W1, b1, W2, b2, Wc, bc):
    raise NotImplementedError("write your pallas kernel here")


# speedup vs baseline: 22.2650x; 22.2650x over previous
import functools

import jax
import jax.numpy as jnp
from jax import lax
from jax.experimental import pallas as pl
from jax.experimental.pallas import tpu as pltpu
from jax.experimental.pallas import tpu_sc as plsc

N_NODES = 10000
NUM_SEGS = 64
NPAD = 10240
NUM_CORES = 2
NUM_SUBCORES = 16
NW = NUM_CORES * NUM_SUBCORES
CHUNK = 128
K_CHUNKS = 79
EDGES_PER_TILE = K_CHUNKS * CHUNK
EPAD = NW * EDGES_PER_TILE
ROWS_PER_SUBCORE = NPAD // NUM_SUBCORES

_mesh = plsc.VectorSubcoreMesh(
    core_axis_name="c", subcore_axis_name="s",
    num_cores=NUM_CORES, num_subcores=NUM_SUBCORES)

_sc_params = pltpu.CompilerParams(needs_layout_passes=False,
                                  use_tc_tiling_on_sc=False)


@functools.partial(
    pl.kernel,
    out_type=jax.ShapeDtypeStruct((NW, NPAD), jnp.float32),
    mesh=_mesh,
    compiler_params=_sc_params,
    scratch_types=[
        pltpu.VMEM((NPAD,), jnp.float32),
        pltpu.VMEM((EDGES_PER_TILE,), jnp.int32),
    ],
)
def _deg_kernel(dst_hbm, out_hbm, hist, dstv):
    c = lax.axis_index("c")
    s = lax.axis_index("s")
    wid = s * NUM_CORES + c

    zero16 = jnp.zeros((16,), jnp.float32)

    def zero_body(i, carry):
        hist[pl.ds(i * 16, 16)] = zero16
        return carry

    lax.fori_loop(0, NPAD // 16, zero_body, 0)

    pltpu.sync_copy(dst_hbm.at[wid], dstv)

    ones16 = jnp.full((16,), 1.0, jnp.float32)

    def body(k, carry):
        idx = dstv[pl.ds(k * 16, 16)]
        plsc.addupdate_scatter(hist, [idx], ones16)
        return carry

    lax.fori_loop(0, EDGES_PER_TILE // 16, body, 0)

    pltpu.sync_copy(hist, out_hbm.at[wid])


def _make_agg(D):
    @functools.partial(
        pl.kernel,
        out_type=jax.ShapeDtypeStruct((NUM_CORES, NPAD, D), jnp.float32),
        mesh=_mesh,
        compiler_params=_sc_params,
        scratch_types=[
            pltpu.VMEM((K_CHUNKS, CHUNK), jnp.int32),
            pltpu.VMEM((K_CHUNKS, CHUNK), jnp.int32),
            pltpu.VMEM((CHUNK, D), jnp.float32),
            pltpu.VMEM_SHARED((NPAD, D), jnp.float32),
            pltpu.SemaphoreType.DMA,
        ],
    )
    def agg(table_hbm, src_hbm, dst_hbm, zeros_hbm, out_hbm,
            srcv, dstv, rows, acc, sem):
        c = lax.axis_index("c")
        s = lax.axis_index("s")
        wid = s * NUM_CORES + c
        base = s * ROWS_PER_SUBCORE

        for off in range(0, ROWS_PER_SUBCORE, CHUNK):
            pltpu.sync_copy(zeros_hbm, acc.at[pl.ds(base + off, CHUNK)])

        pltpu.sync_copy(src_hbm.at[wid], srcv)
        pltpu.sync_copy(dst_hbm.at[wid], dstv)
        plsc.subcore_barrier()

        def body(j, carry):
            pltpu.async_copy(table_hbm.at[srcv.at[j]], rows, sem).wait()
            pltpu.sync_copy(rows, acc.at[dstv.at[j]], add=True)
            return carry

        lax.fori_loop(0, K_CHUNKS, body, 0)

        plsc.subcore_barrier()
        pltpu.sync_copy(acc.at[pl.ds(base, ROWS_PER_SUBCORE)],
                        out_hbm.at[c].at[pl.ds(base, ROWS_PER_SUBCORE)])

    return agg


_agg64 = _make_agg(64)
_agg32 = _make_agg(32)


def _dense1_body(parts_ref, x_ref, w1_ref, dis_ref, h1s_ref):
    deg = jnp.sum(parts_ref[...], axis=0).reshape(NPAD, 1) + 1.0
    row = lax.broadcasted_iota(jnp.int32, (NPAD, 1), 0)
    dis = jnp.where(row < N_NODES, lax.rsqrt(deg), 0.0)
    dis_ref[...] = dis
    h1 = jnp.dot(x_ref[...], w1_ref[...], preferred_element_type=jnp.float32)
    h1s_ref[...] = h1 * dis


def _dense2_body(p_ref, h1s_ref, dis_ref, b1_ref, w2_ref, h2s_ref):
    agg = p_ref[0] + p_ref[1] + h1s_ref[...]
    out1 = jnp.maximum(agg * dis_ref[...] + b1_ref[...], 0.0)
    h2 = jnp.dot(out1, w2_ref[...], preferred_element_type=jnp.float32)
    h2s_ref[...] = h2 * dis_ref[...]


def _final_body(p_ref, h2s_ref, dis_ref, b2_ref, batch_ref, wc_ref, bc_ref,
                out_ref):
    out2 = (p_ref[0] + p_ref[1] + h2s_ref[...]) * dis_ref[...] + b2_ref[...]
    g = lax.broadcasted_iota(jnp.int32, (NUM_SEGS, NPAD), 0)
    b = jnp.broadcast_to(batch_ref[...], (NUM_SEGS, NPAD))
    onehot = jnp.where(b == g, 1.0, 0.0)
    sums = jnp.dot(onehot, out2, preferred_element_type=jnp.float32)
    counts = jnp.sum(onehot, axis=1, keepdims=True)
    pooled = sums / jnp.maximum(counts, 1.0)
    out_ref[...] = (jnp.dot(pooled, wc_ref[...],
                            preferred_element_type=jnp.float32) + bc_ref[...])


def kernel(x, edge_index, batch, W1, b1, W2, b2, Wc, bc):
    src = edge_index[0]
    dst = edge_index[1]
    e = src.shape[0]
    pad = jnp.full((EPAD - e,), N_NODES, jnp.int32)
    src_p = jnp.concatenate([src, pad]).reshape(NW, K_CHUNKS, CHUNK)
    dst_p = jnp.concatenate([dst, pad]).reshape(NW, K_CHUNKS, CHUNK)
    dst_flat = dst_p.reshape(NW, EDGES_PER_TILE)
    x_pad = jnp.pad(x, ((0, NPAD - N_NODES), (0, 0)))
    batch_pad = jnp.pad(batch, (0, NPAD - N_NODES),
                        constant_values=NUM_SEGS).reshape(1, NPAD)
    zeros64 = jnp.zeros((CHUNK, 64), jnp.float32)
    zeros32 = jnp.zeros((CHUNK, 32), jnp.float32)

    parts = _deg_kernel(dst_flat)

    dis, h1s = pl.pallas_call(
        _dense1_body,
        out_shape=(jax.ShapeDtypeStruct((NPAD, 1), jnp.float32),
                   jax.ShapeDtypeStruct((NPAD, 64), jnp.float32)),
    )(parts, x_pad, W1)

    agg1 = _agg64(h1s, src_p, dst_p, zeros64)

    h2s = pl.pallas_call(
        _dense2_body,
        out_shape=jax.ShapeDtypeStruct((NPAD, 32), jnp.float32),
    )(agg1, h1s, dis, b1.reshape(1, 64), W2)

    agg2 = _agg32(h2s, src_p, dst_p, zeros32)

    out = pl.pallas_call(
        _final_body,
        out_shape=jax.ShapeDtypeStruct((NUM_SEGS, 16), jnp.float32),
    )(agg2, h2s, dis, b2.reshape(1, 32), batch_pad, Wc, bc.reshape(1, 16))
    return out


# speedup vs baseline: 24.3641x; 1.0943x over previous
import functools

import jax
import jax.numpy as jnp
from jax import lax
from jax.experimental import pallas as pl
from jax.experimental.pallas import tpu as pltpu
from jax.experimental.pallas import tpu_sc as plsc

N_NODES = 10000
NUM_SEGS = 64
NPAD = 10240
NUM_CORES = 2
NUM_SUBCORES = 16
NW = NUM_CORES * NUM_SUBCORES
CHUNK = 128
K_CHUNKS = 80
NBUF = 8
EDGES_PER_TILE = K_CHUNKS * CHUNK
EPAD = NW * EDGES_PER_TILE
ROWS_PER_SUBCORE = NPAD // NUM_SUBCORES

_mesh = plsc.VectorSubcoreMesh(
    core_axis_name="c", subcore_axis_name="s",
    num_cores=NUM_CORES, num_subcores=NUM_SUBCORES)

_sc_params = pltpu.CompilerParams(needs_layout_passes=False,
                                  use_tc_tiling_on_sc=False)


@functools.partial(
    pl.kernel,
    out_type=jax.ShapeDtypeStruct((NW, NPAD), jnp.float32),
    mesh=_mesh,
    compiler_params=_sc_params,
    scratch_types=[
        pltpu.VMEM((NPAD,), jnp.float32),
        pltpu.VMEM((EDGES_PER_TILE,), jnp.int32),
    ],
)
def _deg_kernel(dst_hbm, out_hbm, hist, dstv):
    c = lax.axis_index("c")
    s = lax.axis_index("s")
    wid = s * NUM_CORES + c

    zero16 = jnp.zeros((16,), jnp.float32)

    def zero_body(i, carry):
        hist[pl.ds(i * 16, 16)] = zero16
        return carry

    lax.fori_loop(0, NPAD // 16, zero_body, 0)

    pltpu.sync_copy(dst_hbm.at[wid], dstv)

    ones16 = jnp.full((16,), 1.0, jnp.float32)

    def body(k, carry):
        idx = dstv[pl.ds(k * 16, 16)]
        plsc.addupdate_scatter(hist, [idx], ones16)
        return carry

    lax.fori_loop(0, EDGES_PER_TILE // 16, body, 0)

    pltpu.sync_copy(hist, out_hbm.at[wid])


def _make_agg(D):
    @functools.partial(
        pl.kernel,
        out_type=jax.ShapeDtypeStruct((NUM_CORES, NPAD, D), jnp.float32),
        mesh=_mesh,
        compiler_params=_sc_params,
        scratch_types=(
            [pltpu.VMEM((K_CHUNKS, CHUNK), jnp.int32),
             pltpu.VMEM((K_CHUNKS, CHUNK), jnp.int32)]
            + [pltpu.VMEM((CHUNK, D), jnp.float32) for _ in range(NBUF)]
            + [pltpu.SemaphoreType.DMA for _ in range(NBUF)]
            + [pltpu.VMEM_SHARED((NPAD, D), jnp.float32)]
        ),
    )
    def agg(table_hbm, src_hbm, dst_hbm, zeros_hbm, out_hbm,
            srcv, dstv, *rest):
        rows = rest[:NBUF]
        sems = rest[NBUF:2 * NBUF]
        acc = rest[2 * NBUF]
        c = lax.axis_index("c")
        s = lax.axis_index("s")
        wid = s * NUM_CORES + c
        base = s * ROWS_PER_SUBCORE

        for off in range(0, ROWS_PER_SUBCORE, CHUNK):
            pltpu.sync_copy(zeros_hbm, acc.at[pl.ds(base + off, CHUNK)])

        pltpu.sync_copy(src_hbm.at[wid], srcv)
        pltpu.sync_copy(dst_hbm.at[wid], dstv)
        plsc.subcore_barrier()

        for b in range(NBUF):
            pltpu.async_copy(table_hbm.at[srcv.at[b]], rows[b], sems[b])

        def body(i, carry):
            for b in range(NBUF):
                j = i * NBUF + b
                pltpu.make_async_copy(
                    table_hbm.at[srcv.at[j]], rows[b], sems[b]).wait()
                pltpu.sync_copy(rows[b], acc.at[dstv.at[j]], add=True)
                pltpu.async_copy(
                    table_hbm.at[srcv.at[j + NBUF]], rows[b], sems[b])
            return carry

        lax.fori_loop(0, K_CHUNKS // NBUF - 1, body, 0)

        for b in range(NBUF):
            j = K_CHUNKS - NBUF + b
            pltpu.make_async_copy(
                table_hbm.at[srcv.at[j]], rows[b], sems[b]).wait()
            pltpu.sync_copy(rows[b], acc.at[dstv.at[j]], add=True)

        plsc.subcore_barrier()
        pltpu.sync_copy(acc.at[pl.ds(base, ROWS_PER_SUBCORE)],
                        out_hbm.at[c].at[pl.ds(base, ROWS_PER_SUBCORE)])

    return agg


_agg64 = _make_agg(64)
_agg32 = _make_agg(32)


def _dense1_body(parts_ref, x_ref, w1_ref, dis_ref, h1s_ref):
    deg = jnp.sum(parts_ref[...], axis=0).reshape(NPAD, 1) + 1.0
    row = lax.broadcasted_iota(jnp.int32, (NPAD, 1), 0)
    dis = jnp.where(row < N_NODES, lax.rsqrt(deg), 0.0)
    dis_ref[...] = dis
    h1 = jnp.dot(x_ref[...], w1_ref[...], preferred_element_type=jnp.float32)
    h1s_ref[...] = h1 * dis


def _dense2_body(p_ref, h1s_ref, dis_ref, b1_ref, w2_ref, h2s_ref):
    agg = p_ref[0] + p_ref[1] + h1s_ref[...]
    out1 = jnp.maximum(agg * dis_ref[...] + b1_ref[...], 0.0)
    h2 = jnp.dot(out1, w2_ref[...], preferred_element_type=jnp.float32)
    h2s_ref[...] = h2 * dis_ref[...]


def _final_body(p_ref, h2s_ref, dis_ref, b2_ref, batch_ref, wc_ref, bc_ref,
                out_ref):
    out2 = (p_ref[0] + p_ref[1] + h2s_ref[...]) * dis_ref[...] + b2_ref[...]
    g = lax.broadcasted_iota(jnp.int32, (NUM_SEGS, NPAD), 0)
    b = jnp.broadcast_to(batch_ref[...], (NUM_SEGS, NPAD))
    onehot = jnp.where(b == g, 1.0, 0.0)
    sums = jnp.dot(onehot, out2, preferred_element_type=jnp.float32)
    counts = jnp.sum(onehot, axis=1, keepdims=True)
    pooled = sums / jnp.maximum(counts, 1.0)
    out_ref[...] = (jnp.dot(pooled, wc_ref[...],
                            preferred_element_type=jnp.float32) + bc_ref[...])


def kernel(x, edge_index, batch, W1, b1, W2, b2, Wc, bc):
    src = edge_index[0]
    dst = edge_index[1]
    e = src.shape[0]
    pad = jnp.full((EPAD - e,), N_NODES, jnp.int32)
    src_p = jnp.concatenate([src, pad]).reshape(NW, K_CHUNKS, CHUNK)
    dst_p = jnp.concatenate([dst, pad]).reshape(NW, K_CHUNKS, CHUNK)
    dst_flat = dst_p.reshape(NW, EDGES_PER_TILE)
    x_pad = jnp.pad(x, ((0, NPAD - N_NODES), (0, 0)))
    batch_pad = jnp.pad(batch, (0, NPAD - N_NODES),
                        constant_values=NUM_SEGS).reshape(1, NPAD)
    zeros64 = jnp.zeros((CHUNK, 64), jnp.float32)
    zeros32 = jnp.zeros((CHUNK, 32), jnp.float32)

    parts = _deg_kernel(dst_flat)

    dis, h1s = pl.pallas_call(
        _dense1_body,
        out_shape=(jax.ShapeDtypeStruct((NPAD, 1), jnp.float32),
                   jax.ShapeDtypeStruct((NPAD, 64), jnp.float32)),
    )(parts, x_pad, W1)

    agg1 = _agg64(h1s, src_p, dst_p, zeros64)

    h2s = pl.pallas_call(
        _dense2_body,
        out_shape=jax.ShapeDtypeStruct((NPAD, 32), jnp.float32),
    )(agg1, h1s, dis, b1.reshape(1, 64), W2)

    agg2 = _agg32(h2s, src_p, dst_p, zeros32)

    out = pl.pallas_call(
        _final_body,
        out_shape=jax.ShapeDtypeStruct((NUM_SEGS, 16), jnp.float32),
    )(agg2, h2s, dis, b2.reshape(1, 32), batch_pad, Wc, bc.reshape(1, 16))
    return out


# speedup vs baseline: 37.8620x; 1.5540x over previous
import functools

import jax
import jax.numpy as jnp
from jax import lax
from jax.experimental import pallas as pl
from jax.experimental.pallas import tpu as pltpu
from jax.experimental.pallas import tpu_sc as plsc

N_NODES = 10000
NUM_SEGS = 64
NPAD = 10240
NUM_CORES = 2
NUM_SUBCORES = 16
NW = NUM_CORES * NUM_SUBCORES
CHUNK = 128
K_CHUNKS = 80
NBUF = 4
GROUPS = K_CHUNKS // NBUF
EDGES_PER_TILE = K_CHUNKS * CHUNK
EPAD = NW * EDGES_PER_TILE
ROWS_PER_SUBCORE = NPAD // NUM_SUBCORES

_mesh = plsc.VectorSubcoreMesh(
    core_axis_name="c", subcore_axis_name="s",
    num_cores=NUM_CORES, num_subcores=NUM_SUBCORES)

_sc_params = pltpu.CompilerParams(needs_layout_passes=False,
                                  use_tc_tiling_on_sc=False)


@functools.partial(
    pl.kernel,
    out_type=jax.ShapeDtypeStruct((NW, NPAD), jnp.float32),
    mesh=_mesh,
    compiler_params=_sc_params,
    scratch_types=[
        pltpu.VMEM((NPAD,), jnp.float32),
        pltpu.VMEM((EDGES_PER_TILE,), jnp.int32),
    ],
)
def _deg_kernel(dst_hbm, out_hbm, hist, dstv):
    c = lax.axis_index("c")
    s = lax.axis_index("s")
    wid = s * NUM_CORES + c

    zero16 = jnp.zeros((16,), jnp.float32)

    def zero_body(i, carry):
        hist[pl.ds(i * 16, 16)] = zero16
        return carry

    lax.fori_loop(0, NPAD // 16, zero_body, 0)

    pltpu.sync_copy(dst_hbm.at[wid], dstv)

    ones16 = jnp.full((16,), 1.0, jnp.float32)

    def body(k, carry):
        idx = dstv[pl.ds(k * 16, 16)]
        plsc.addupdate_scatter(hist, [idx], ones16)
        return carry

    lax.fori_loop(0, EDGES_PER_TILE // 16, body, 0)

    pltpu.sync_copy(hist, out_hbm.at[wid])


def _make_agg(D):
    @functools.partial(
        pl.kernel,
        out_type=jax.ShapeDtypeStruct((NUM_CORES, NPAD, D), jnp.float32),
        mesh=_mesh,
        compiler_params=_sc_params,
        scratch_types=(
            [pltpu.VMEM((NBUF, CHUNK), jnp.int32) for _ in range(4)]
            + [pltpu.VMEM((CHUNK, D), jnp.float32) for _ in range(NBUF)]
            + [pltpu.SemaphoreType.DMA for _ in range(NBUF + 2)]
            + [pltpu.VMEM_SHARED((NPAD, D), jnp.float32),
               pltpu.VMEM_SHARED((NPAD, D), jnp.float32)]
        ),
    )
    def agg(table_hbm, src_hbm, dst_hbm, zeros_hbm, out_hbm, *rest):
        srcg = rest[0:2]
        dstg = rest[2:4]
        rows = rest[4:4 + NBUF]
        sems = rest[4 + NBUF:4 + 2 * NBUF]
        semi = rest[4 + 2 * NBUF:6 + 2 * NBUF]
        acc = rest[6 + 2 * NBUF]
        table_sp = rest[7 + 2 * NBUF]
        c = lax.axis_index("c")
        s = lax.axis_index("s")
        wid = s * NUM_CORES + c
        base = s * ROWS_PER_SUBCORE

        for off in range(0, ROWS_PER_SUBCORE, CHUNK):
            pltpu.sync_copy(zeros_hbm, acc.at[pl.ds(base + off, CHUNK)])

        pltpu.sync_copy(table_hbm.at[pl.ds(base, ROWS_PER_SUBCORE)],
                        table_sp.at[pl.ds(base, ROWS_PER_SUBCORE)])

        def issue_idx(g, st):
            pltpu.async_copy(
                src_hbm.at[wid].at[pl.ds(g * NBUF, NBUF)], srcg[st], semi[st])
            pltpu.async_copy(
                dst_hbm.at[wid].at[pl.ds(g * NBUF, NBUF)], dstg[st], semi[st])

        def wait_idx(g, st):
            pltpu.make_async_copy(
                src_hbm.at[wid].at[pl.ds(g * NBUF, NBUF)], srcg[st],
                semi[st]).wait()
            pltpu.make_async_copy(
                dst_hbm.at[wid].at[pl.ds(g * NBUF, NBUF)], dstg[st],
                semi[st]).wait()

        issue_idx(0, 0)
        plsc.subcore_barrier()

        def run_group(g, st):
            for b in range(NBUF):
                pltpu.async_copy(table_sp.at[srcg[st].at[b]], rows[b], sems[b])
            for b in range(NBUF):
                pltpu.make_async_copy(
                    table_sp.at[srcg[st].at[b]], rows[b], sems[b]).wait()
                pltpu.sync_copy(rows[b], acc.at[dstg[st].at[b]], add=True)

        def body(p, carry):
            wait_idx(2 * p, 0)
            issue_idx(2 * p + 1, 1)
            run_group(2 * p, 0)
            wait_idx(2 * p + 1, 1)

            @pl.when(p < GROUPS // 2 - 1)
            def _():
                issue_idx(2 * p + 2, 0)

            run_group(2 * p + 1, 1)
            return carry

        lax.fori_loop(0, GROUPS // 2, body, 0)

        plsc.subcore_barrier()
        pltpu.sync_copy(acc.at[pl.ds(base, ROWS_PER_SUBCORE)],
                        out_hbm.at[c].at[pl.ds(base, ROWS_PER_SUBCORE)])

    return agg


_agg64 = _make_agg(64)
_agg32 = _make_agg(32)


def _dense1_body(parts_ref, x_ref, w1_ref, dis_ref, h1s_ref):
    deg = jnp.sum(parts_ref[...], axis=0).reshape(NPAD, 1) + 1.0
    row = lax.broadcasted_iota(jnp.int32, (NPAD, 1), 0)
    dis = jnp.where(row < N_NODES, lax.rsqrt(deg), 0.0)
    dis_ref[...] = dis
    h1 = jnp.dot(x_ref[...], w1_ref[...], preferred_element_type=jnp.float32)
    h1s_ref[...] = h1 * dis


def _dense2_body(p_ref, h1s_ref, dis_ref, b1_ref, w2_ref, h2s_ref):
    agg = p_ref[0] + p_ref[1] + h1s_ref[...]
    out1 = jnp.maximum(agg * dis_ref[...] + b1_ref[...], 0.0)
    h2 = jnp.dot(out1, w2_ref[...], preferred_element_type=jnp.float32)
    h2s_ref[...] = h2 * dis_ref[...]


def _final_body(p_ref, h2s_ref, dis_ref, b2_ref, batch_ref, wc_ref, bc_ref,
                out_ref):
    out2 = (p_ref[0] + p_ref[1] + h2s_ref[...]) * dis_ref[...] + b2_ref[...]
    g = lax.broadcasted_iota(jnp.int32, (NUM_SEGS, NPAD), 0)
    b = jnp.broadcast_to(batch_ref[...], (NUM_SEGS, NPAD))
    onehot = jnp.where(b == g, 1.0, 0.0)
    sums = jnp.dot(onehot, out2, preferred_element_type=jnp.float32)
    counts = jnp.sum(onehot, axis=1, keepdims=True)
    pooled = sums / jnp.maximum(counts, 1.0)
    out_ref[...] = (jnp.dot(pooled, wc_ref[...],
                            preferred_element_type=jnp.float32) + bc_ref[...])


def kernel(x, edge_index, batch, W1, b1, W2, b2, Wc, bc):
    src = edge_index[0]
    dst = edge_index[1]
    e = src.shape[0]
    pad = jnp.full((EPAD - e,), N_NODES, jnp.int32)
    src_p = jnp.concatenate([src, pad]).reshape(NW, K_CHUNKS, CHUNK)
    dst_p = jnp.concatenate([dst, pad]).reshape(NW, K_CHUNKS, CHUNK)
    dst_flat = dst_p.reshape(NW, EDGES_PER_TILE)
    x_pad = jnp.pad(x, ((0, NPAD - N_NODES), (0, 0)))
    batch_pad = jnp.pad(batch, (0, NPAD - N_NODES),
                        constant_values=NUM_SEGS).reshape(1, NPAD)
    zeros64 = jnp.zeros((CHUNK, 64), jnp.float32)
    zeros32 = jnp.zeros((CHUNK, 32), jnp.float32)

    parts = _deg_kernel(dst_flat)

    dis, h1s = pl.pallas_call(
        _dense1_body,
        out_shape=(jax.ShapeDtypeStruct((NPAD, 1), jnp.float32),
                   jax.ShapeDtypeStruct((NPAD, 64), jnp.float32)),
    )(parts, x_pad, W1)

    agg1 = _agg64(h1s, src_p, dst_p, zeros64)

    h2s = pl.pallas_call(
        _dense2_body,
        out_shape=jax.ShapeDtypeStruct((NPAD, 32), jnp.float32),
    )(agg1, h1s, dis, b1.reshape(1, 64), W2)

    agg2 = _agg32(h2s, src_p, dst_p, zeros32)

    out = pl.pallas_call(
        _final_body,
        out_shape=jax.ShapeDtypeStruct((NUM_SEGS, 16), jnp.float32),
    )(agg2, h2s, dis, b2.reshape(1, 32), batch_pad, Wc, bc.reshape(1, 16))
    return out


# speedup vs baseline: 45.0831x; 1.1907x over previous
import functools

import jax
import jax.numpy as jnp
from jax import lax
from jax.experimental import pallas as pl
from jax.experimental.pallas import tpu as pltpu
from jax.experimental.pallas import tpu_sc as plsc

N_NODES = 10000
NUM_SEGS = 64
NUM_CORES = 2
NUM_SUBCORES = 16
NW = NUM_CORES * NUM_SUBCORES
CHUNK = 125
K_CHUNKS = 80
NBUF = 4
GROUPS = K_CHUNKS // NBUF
EDGES_PER_TILE = K_CHUNKS * CHUNK
ROWS_PER_SUBCORE = N_NODES // NUM_SUBCORES

_mesh = plsc.VectorSubcoreMesh(
    core_axis_name="c", subcore_axis_name="s",
    num_cores=NUM_CORES, num_subcores=NUM_SUBCORES)

_sc_params = pltpu.CompilerParams(needs_layout_passes=False,
                                  use_tc_tiling_on_sc=False)


@functools.partial(
    pl.kernel,
    out_type=jax.ShapeDtypeStruct((NW, N_NODES), jnp.float32),
    mesh=_mesh,
    compiler_params=_sc_params,
    scratch_types=[
        pltpu.VMEM((N_NODES,), jnp.float32),
        pltpu.VMEM((EDGES_PER_TILE,), jnp.int32),
    ],
)
def _deg_kernel(dst_hbm, out_hbm, hist, dstv):
    c = lax.axis_index("c")
    s = lax.axis_index("s")
    wid = s * NUM_CORES + c

    zero16 = jnp.zeros((16,), jnp.float32)

    def zero_body(i, carry):
        hist[pl.ds(i * 16, 16)] = zero16
        return carry

    lax.fori_loop(0, N_NODES // 16, zero_body, 0)

    pltpu.sync_copy(dst_hbm.at[wid], dstv)

    ones16 = jnp.full((16,), 1.0, jnp.float32)

    def body(k, carry):
        idx = dstv[pl.ds(k * 16, 16)]
        plsc.addupdate_scatter(hist, [idx], ones16)
        return carry

    lax.fori_loop(0, EDGES_PER_TILE // 16, body, 0)

    pltpu.sync_copy(hist, out_hbm.at[wid])


def _make_agg(D):
    @functools.partial(
        pl.kernel,
        out_type=jax.ShapeDtypeStruct((NUM_CORES, N_NODES, D), jnp.float32),
        mesh=_mesh,
        compiler_params=_sc_params,
        scratch_types=(
            [pltpu.VMEM((NBUF, CHUNK), jnp.int32) for _ in range(4)]
            + [pltpu.VMEM((CHUNK, D), jnp.float32) for _ in range(NBUF)]
            + [pltpu.SemaphoreType.DMA for _ in range(NBUF + 2)]
            + [pltpu.VMEM_SHARED((N_NODES, D), jnp.float32),
               pltpu.VMEM_SHARED((N_NODES, D), jnp.float32)]
        ),
    )
    def agg(table_hbm, src_hbm, dst_hbm, zeros_hbm, out_hbm, *rest):
        srcg = rest[0:2]
        dstg = rest[2:4]
        rows = rest[4:4 + NBUF]
        sems = rest[4 + NBUF:4 + 2 * NBUF]
        semi = rest[4 + 2 * NBUF:6 + 2 * NBUF]
        acc = rest[6 + 2 * NBUF]
        table_sp = rest[7 + 2 * NBUF]
        c = lax.axis_index("c")
        s = lax.axis_index("s")
        wid = s * NUM_CORES + c
        base = s * ROWS_PER_SUBCORE

        for off in range(0, ROWS_PER_SUBCORE, CHUNK):
            pltpu.sync_copy(zeros_hbm, acc.at[pl.ds(base + off, CHUNK)])

        pltpu.sync_copy(table_hbm.at[pl.ds(base, ROWS_PER_SUBCORE)],
                        table_sp.at[pl.ds(base, ROWS_PER_SUBCORE)])

        def issue_idx(g, st):
            pltpu.async_copy(
                src_hbm.at[wid].at[pl.ds(g * NBUF, NBUF)], srcg[st], semi[st])
            pltpu.async_copy(
                dst_hbm.at[wid].at[pl.ds(g * NBUF, NBUF)], dstg[st], semi[st])

        def wait_idx(g, st):
            pltpu.make_async_copy(
                src_hbm.at[wid].at[pl.ds(g * NBUF, NBUF)], srcg[st],
                semi[st]).wait()
            pltpu.make_async_copy(
                dst_hbm.at[wid].at[pl.ds(g * NBUF, NBUF)], dstg[st],
                semi[st]).wait()

        issue_idx(0, 0)
        plsc.subcore_barrier()

        def run_group(g, st):
            for b in range(NBUF):
                pltpu.async_copy(table_sp.at[srcg[st].at[b]], rows[b], sems[b])
            for b in range(NBUF):
                pltpu.make_async_copy(
                    table_sp.at[srcg[st].at[b]], rows[b], sems[b]).wait()
                pltpu.sync_copy(rows[b], acc.at[dstg[st].at[b]], add=True)

        def body(p, carry):
            wait_idx(2 * p, 0)
            issue_idx(2 * p + 1, 1)
            run_group(2 * p, 0)
            wait_idx(2 * p + 1, 1)

            @pl.when(p < GROUPS // 2 - 1)
            def _():
                issue_idx(2 * p + 2, 0)

            run_group(2 * p + 1, 1)
            return carry

        lax.fori_loop(0, GROUPS // 2, body, 0)

        plsc.subcore_barrier()
        pltpu.sync_copy(acc.at[pl.ds(base, ROWS_PER_SUBCORE)],
                        out_hbm.at[c].at[pl.ds(base, ROWS_PER_SUBCORE)])

    return agg


_agg64 = _make_agg(64)
_agg32 = _make_agg(32)


def _dense1_body(parts_ref, x_ref, w1_ref, dis_ref, h1s_ref):
    deg = jnp.sum(parts_ref[...], axis=0).reshape(N_NODES, 1) + 1.0
    dis = lax.rsqrt(deg)
    dis_ref[...] = dis
    h1 = jnp.dot(x_ref[...], w1_ref[...], preferred_element_type=jnp.float32)
    h1s_ref[...] = h1 * dis


def _dense2_body(p_ref, h1s_ref, dis_ref, b1_ref, w2_ref, h2s_ref):
    agg = p_ref[0] + p_ref[1] + h1s_ref[...]
    out1 = jnp.maximum(agg * dis_ref[...] + b1_ref[...], 0.0)
    h2 = jnp.dot(out1, w2_ref[...], preferred_element_type=jnp.float32)
    h2s_ref[...] = h2 * dis_ref[...]


def _final_body(p_ref, h2s_ref, dis_ref, b2_ref, batch_ref, wc_ref, bc_ref,
                out_ref):
    out2 = (p_ref[0] + p_ref[1] + h2s_ref[...]) * dis_ref[...] + b2_ref[...]
    g = lax.broadcasted_iota(jnp.int32, (NUM_SEGS, N_NODES), 0)
    b = jnp.broadcast_to(batch_ref[...], (NUM_SEGS, N_NODES))
    onehot = jnp.where(b == g, 1.0, 0.0)
    sums = jnp.dot(onehot, out2, preferred_element_type=jnp.float32)
    counts = jnp.sum(onehot, axis=1, keepdims=True)
    pooled = sums / jnp.maximum(counts, 1.0)
    out_ref[...] = (jnp.dot(pooled, wc_ref[...],
                            preferred_element_type=jnp.float32) + bc_ref[...])


def kernel(x, edge_index, batch, W1, b1, W2, b2, Wc, bc):
    src_p = edge_index[0].reshape(NW, K_CHUNKS, CHUNK)
    dst_p = edge_index[1].reshape(NW, K_CHUNKS, CHUNK)
    dst_flat = edge_index[1].reshape(NW, EDGES_PER_TILE)
    batch2 = batch.reshape(1, N_NODES)
    zeros64 = jnp.zeros((CHUNK, 64), jnp.float32)
    zeros32 = jnp.zeros((CHUNK, 32), jnp.float32)

    parts = _deg_kernel(dst_flat)

    dis, h1s = pl.pallas_call(
        _dense1_body,
        out_shape=(jax.ShapeDtypeStruct((N_NODES, 1), jnp.float32),
                   jax.ShapeDtypeStruct((N_NODES, 64), jnp.float32)),
    )(parts, x, W1)

    agg1 = _agg64(h1s, src_p, dst_p, zeros64)

    h2s = pl.pallas_call(
        _dense2_body,
        out_shape=jax.ShapeDtypeStruct((N_NODES, 32), jnp.float32),
    )(agg1, h1s, dis, b1.reshape(1, 64), W2)

    agg2 = _agg32(h2s, src_p, dst_p, zeros32)

    out = pl.pallas_call(
        _final_body,
        out_shape=jax.ShapeDtypeStruct((NUM_SEGS, 16), jnp.float32),
    )(agg2, h2s, dis, b2.reshape(1, 32), batch2, Wc, bc.reshape(1, 16))
    return out


# speedup vs baseline: 46.2922x; 1.0268x over previous
import functools

import jax
import jax.numpy as jnp
from jax import lax
from jax.experimental import pallas as pl
from jax.experimental.pallas import tpu as pltpu
from jax.experimental.pallas import tpu_sc as plsc

N_NODES = 10000
NUM_SEGS = 64
NUM_CORES = 2
NUM_SUBCORES = 16
NW = NUM_CORES * NUM_SUBCORES
CHUNK = 125
K_CHUNKS = 80
NBUF = 4
GROUPS = K_CHUNKS // NBUF
EDGES_PER_TILE = K_CHUNKS * CHUNK
ROWS_PER_SUBCORE = N_NODES // NUM_SUBCORES

_mesh = plsc.VectorSubcoreMesh(
    core_axis_name="c", subcore_axis_name="s",
    num_cores=NUM_CORES, num_subcores=NUM_SUBCORES)

_sc_params = pltpu.CompilerParams(needs_layout_passes=False,
                                  use_tc_tiling_on_sc=False)


@functools.partial(
    pl.kernel,
    out_type=jax.ShapeDtypeStruct((NW, N_NODES), jnp.float32),
    mesh=_mesh,
    compiler_params=_sc_params,
    scratch_types=[
        pltpu.VMEM((N_NODES,), jnp.float32),
        pltpu.VMEM((EDGES_PER_TILE,), jnp.int32),
        pltpu.SemaphoreType.DMA,
    ],
)
def _deg_kernel(dst_hbm, out_hbm, hist, dstv, dsem):
    c = lax.axis_index("c")
    s = lax.axis_index("s")
    wid = s * NUM_CORES + c

    cp = pltpu.async_copy(dst_hbm.at[wid], dstv, dsem)

    zero16 = jnp.zeros((16,), jnp.float32)

    def zero_body(i, carry):
        hist[pl.ds(i * 16, 16)] = zero16
        return carry

    lax.fori_loop(0, N_NODES // 16, zero_body, 0)

    cp.wait()

    ones16 = jnp.full((16,), 1.0, jnp.float32)

    def body(k, carry):
        idx = dstv[pl.ds(k * 16, 16)]
        plsc.addupdate_scatter(hist, [idx], ones16)
        return carry

    lax.fori_loop(0, EDGES_PER_TILE // 16, body, 0)

    pltpu.sync_copy(hist, out_hbm.at[wid])


def _make_agg(D):
    @functools.partial(
        pl.kernel,
        out_type=jax.ShapeDtypeStruct((NUM_CORES, N_NODES, D), jnp.float32),
        mesh=_mesh,
        compiler_params=_sc_params,
        scratch_types=(
            [pltpu.VMEM((NBUF, CHUNK), jnp.int32) for _ in range(4)]
            + [pltpu.VMEM((CHUNK, D), jnp.float32) for _ in range(NBUF)]
            + [pltpu.SemaphoreType.DMA for _ in range(NBUF + 3)]
            + [pltpu.VMEM_SHARED((N_NODES, D), jnp.float32),
               pltpu.VMEM_SHARED((N_NODES, D), jnp.float32)]
        ),
    )
    def agg(table_hbm, src_hbm, dst_hbm, zeros_hbm, out_hbm, *rest):
        srcg = rest[0:2]
        dstg = rest[2:4]
        rows = rest[4:4 + NBUF]
        sems = rest[4 + NBUF:4 + 2 * NBUF]
        semi = rest[4 + 2 * NBUF:6 + 2 * NBUF]
        zsem = rest[6 + 2 * NBUF]
        acc = rest[7 + 2 * NBUF]
        table_sp = rest[8 + 2 * NBUF]
        c = lax.axis_index("c")
        s = lax.axis_index("s")
        wid = s * NUM_CORES + c
        base = s * ROWS_PER_SUBCORE

        for off in range(0, ROWS_PER_SUBCORE, CHUNK):
            pltpu.async_copy(zeros_hbm, acc.at[pl.ds(base + off, CHUNK)],
                             zsem)
        pltpu.async_copy(table_hbm.at[pl.ds(base, ROWS_PER_SUBCORE)],
                         table_sp.at[pl.ds(base, ROWS_PER_SUBCORE)], zsem)

        def issue_idx(g, st):
            pltpu.async_copy(
                src_hbm.at[wid].at[pl.ds(g * NBUF, NBUF)], srcg[st], semi[st])
            pltpu.async_copy(
                dst_hbm.at[wid].at[pl.ds(g * NBUF, NBUF)], dstg[st], semi[st])

        def wait_idx(g, st):
            pltpu.make_async_copy(
                src_hbm.at[wid].at[pl.ds(g * NBUF, NBUF)], srcg[st],
                semi[st]).wait()
            pltpu.make_async_copy(
                dst_hbm.at[wid].at[pl.ds(g * NBUF, NBUF)], dstg[st],
                semi[st]).wait()

        issue_idx(0, 0)
        for off in range(0, ROWS_PER_SUBCORE, CHUNK):
            pltpu.make_async_copy(
                zeros_hbm, acc.at[pl.ds(base + off, CHUNK)], zsem).wait()
        pltpu.make_async_copy(
            table_hbm.at[pl.ds(base, ROWS_PER_SUBCORE)],
            table_sp.at[pl.ds(base, ROWS_PER_SUBCORE)], zsem).wait()
        plsc.subcore_barrier()

        def run_group(g, st):
            for b in range(NBUF):
                pltpu.async_copy(table_sp.at[srcg[st].at[b]], rows[b], sems[b])
            for b in range(NBUF):
                pltpu.make_async_copy(
                    table_sp.at[srcg[st].at[b]], rows[b], sems[b]).wait()
                pltpu.sync_copy(rows[b], acc.at[dstg[st].at[b]], add=True)

        def body(p, carry):
            wait_idx(2 * p, 0)
            issue_idx(2 * p + 1, 1)
            run_group(2 * p, 0)
            wait_idx(2 * p + 1, 1)

            @pl.when(p < GROUPS // 2 - 1)
            def _():
                issue_idx(2 * p + 2, 0)

            run_group(2 * p + 1, 1)
            return carry

        lax.fori_loop(0, GROUPS // 2, body, 0)

        plsc.subcore_barrier()
        pltpu.sync_copy(acc.at[pl.ds(base, ROWS_PER_SUBCORE)],
                        out_hbm.at[c].at[pl.ds(base, ROWS_PER_SUBCORE)])

    return agg


_agg64 = _make_agg(64)
_agg32 = _make_agg(32)


def _dense1_body(parts_ref, x_ref, w1_ref, dis_ref, h1s_ref):
    deg = jnp.sum(parts_ref[...], axis=0).reshape(N_NODES, 1) + 1.0
    dis = lax.rsqrt(deg)
    dis_ref[...] = dis
    h1 = jnp.dot(x_ref[...], w1_ref[...], preferred_element_type=jnp.float32)
    h1s_ref[...] = h1 * dis


def _dense2_body(p_ref, h1s_ref, dis_ref, b1_ref, w2_ref, h2s_ref):
    agg = p_ref[0] + p_ref[1] + h1s_ref[...]
    out1 = jnp.maximum(agg * dis_ref[...] + b1_ref[...], 0.0)
    h2 = jnp.dot(out1, w2_ref[...], preferred_element_type=jnp.float32)
    h2s_ref[...] = h2 * dis_ref[...]


def _final_body(p_ref, h2s_ref, dis_ref, b2_ref, batch_ref, wc_ref, bc_ref,
                out_ref):
    out2 = (p_ref[0] + p_ref[1] + h2s_ref[...]) * dis_ref[...] + b2_ref[...]
    g = lax.broadcasted_iota(jnp.int32, (NUM_SEGS, N_NODES), 0)
    b = jnp.broadcast_to(batch_ref[...], (NUM_SEGS, N_NODES))
    onehot = jnp.where(b == g, 1.0, 0.0)
    sums = jnp.dot(onehot, out2, preferred_element_type=jnp.float32)
    counts = jnp.sum(onehot, axis=1, keepdims=True)
    pooled = sums / jnp.maximum(counts, 1.0)
    out_ref[...] = (jnp.dot(pooled, wc_ref[...],
                            preferred_element_type=jnp.float32) + bc_ref[...])


def kernel(x, edge_index, batch, W1, b1, W2, b2, Wc, bc):
    src_p = edge_index[0].reshape(NW, K_CHUNKS, CHUNK)
    dst_p = edge_index[1].reshape(NW, K_CHUNKS, CHUNK)
    dst_flat = edge_index[1].reshape(NW, EDGES_PER_TILE)
    batch2 = batch.reshape(1, N_NODES)
    zeros64 = jnp.zeros((CHUNK, 64), jnp.float32)
    zeros32 = jnp.zeros((CHUNK, 32), jnp.float32)

    parts = _deg_kernel(dst_flat)

    dis, h1s = pl.pallas_call(
        _dense1_body,
        out_shape=(jax.ShapeDtypeStruct((N_NODES, 1), jnp.float32),
                   jax.ShapeDtypeStruct((N_NODES, 64), jnp.float32)),
    )(parts, x, W1)

    agg1 = _agg64(h1s, src_p, dst_p, zeros64)

    h2s = pl.pallas_call(
        _dense2_body,
        out_shape=jax.ShapeDtypeStruct((N_NODES, 32), jnp.float32),
    )(agg1, h1s, dis, b1.reshape(1, 64), W2)

    agg2 = _agg32(h2s, src_p, dst_p, zeros32)

    out = pl.pallas_call(
        _final_body,
        out_shape=jax.ShapeDtypeStruct((NUM_SEGS, 16), jnp.float32),
    )(agg2, h2s, dis, b2.reshape(1, 32), batch2, Wc, bc.reshape(1, 16))
    return out
